# baseline (speedup 1.0000x reference)
import functools

import jax
import jax.numpy as jnp
from jax import lax
from jax.experimental import pallas as pl
from jax.experimental.pallas import tpu as pltpu
from jax.experimental.pallas import tpu_sc as plsc

B, L = 1024, 200
NPOS = B * L
K = 16
NCOL = 100001
TSTRIDE = K * NCOL
NW = 32
POS_PER_W = NPOS // NW
CHUNK = 640
NCHUNK = POS_PER_W // CHUNK
GROUPS = CHUNK // 16


def _sc_gather(items, t1, t2, mflat, wflat):
    mesh = plsc.VectorSubcoreMesh(core_axis_name="c", subcore_axis_name="s")

    @functools.partial(
        pl.kernel,
        out_type=[
            jax.ShapeDtypeStruct((NPOS * K,), jnp.float32),
            jax.ShapeDtypeStruct((NPOS * K,), jnp.float32),
        ],
        mesh=mesh,
        compiler_params=pltpu.CompilerParams(needs_layout_passes=False),
        scratch_types=[
            pltpu.VMEM((CHUNK,), jnp.int32),
            pltpu.VMEM((CHUNK,), jnp.int32),
            pltpu.VMEM((CHUNK,), jnp.int32),
            pltpu.VMEM((CHUNK * K,), jnp.int32),
            pltpu.VMEM((CHUNK * K,), jnp.int32),
            pltpu.VMEM((CHUNK * K,), jnp.float32),
            pltpu.VMEM((CHUNK * K,), jnp.float32),
            pltpu.SemaphoreType.DMA,
            pltpu.SemaphoreType.DMA,
        ],
    )
    def body(items_h, t1_h, t2_h, mflat_h, wflat_h, om_h, ow_h,
             items_v, t1_v, t2_v, idx_m, idx_w, gm, gw, sem_m, sem_w):
        wid = lax.axis_index("s") * 2 + lax.axis_index("c")
        lane_tgt0 = lax.iota(jnp.int32, 16) * K

        @pl.loop(0, NCHUNK)
        def _chunk(c):
            base = wid * POS_PER_W + c * CHUNK
            pltpu.sync_copy(items_h.at[pl.ds(base, CHUNK)], items_v)
            pltpu.sync_copy(t1_h.at[pl.ds(base, CHUNK)], t1_v)
            pltpu.sync_copy(t2_h.at[pl.ds(base, CHUNK)], t2_v)

            @pl.loop(0, GROUPS)
            def _group(g):
                it = items_v[pl.ds(g * 16, 16)]
                bm = t1_v[pl.ds(g * 16, 16)] * TSTRIDE + it
                bw = t2_v[pl.ds(g * 16, 16)] * TSTRIDE + it
                goff = g * 256
                for k in range(K):
                    tgt = lane_tgt0 + (goff + k)
                    plsc.store_scatter(idx_m, [tgt], bm + k * NCOL)
                    plsc.store_scatter(idx_w, [tgt], bw + k * NCOL)

            cm = pltpu.async_copy(mflat_h.at[idx_m], gm, sem_m)
            cw = pltpu.async_copy(wflat_h.at[idx_w], gw, sem_w)
            cm.wait()
            cw.wait()
            pltpu.sync_copy(gm, om_h.at[pl.ds(base * K, CHUNK * K)])
            pltpu.sync_copy(gw, ow_h.at[pl.ds(base * K, CHUNK * K)])

    return body(items, t1, t2, mflat, wflat)


def kernel(log_seqs, time1_seqs, time2_seqs, month_pop_table, week_pop_table):
    items = log_seqs.reshape(-1).astype(jnp.int32)
    t1 = time1_seqs.reshape(-1).astype(jnp.int32)
    t2 = time2_seqs.reshape(-1).astype(jnp.int32)
    mflat = month_pop_table.reshape(-1)
    wflat = week_pop_table.reshape(-1)
    out_m, out_w = _sc_gather(items, t1, t2, mflat, wflat)
    return jnp.concatenate(
        [out_m.reshape(B, L, K), out_w.reshape(B, L, K)], axis=2)


# speedup vs baseline: 2.6172x; 2.6172x over previous
import functools

import jax
import jax.numpy as jnp
from jax import lax
from jax.experimental import pallas as pl
from jax.experimental.pallas import tpu as pltpu
from jax.experimental.pallas import tpu_sc as plsc

B, L = 1024, 200
NPOS = B * L
K = 16
NITEM = 100001
NM, NWK = 24, 52
MROWS = 3
WROWS = 7
NW = 32
POS_PER_W = NPOS // NW
CHUNK = 400
NCHUNK = POS_PER_W // CHUNK
GROUPS = CHUNK // 16


def _sc_gather(items, t1, t2, m128, w896):
    mesh = plsc.VectorSubcoreMesh(core_axis_name="c", subcore_axis_name="s")

    @functools.partial(
        pl.kernel,
        out_type=jax.ShapeDtypeStruct((NPOS * 2 * K,), jnp.float32),
        mesh=mesh,
        compiler_params=pltpu.CompilerParams(needs_layout_passes=False),
        scratch_types=[
            pltpu.VMEM((CHUNK,), jnp.int32),
            pltpu.VMEM((CHUNK,), jnp.int32),
            pltpu.VMEM((CHUNK,), jnp.int32),
            pltpu.VMEM((CHUNK,), jnp.int32),
            pltpu.VMEM((CHUNK,), jnp.int32),
            pltpu.VMEM((CHUNK, 128), jnp.float32),
            pltpu.VMEM((CHUNK, 128), jnp.float32),
            pltpu.VMEM((CHUNK * 2 * K,), jnp.float32),
            pltpu.SemaphoreType.DMA,
            pltpu.SemaphoreType.DMA,
        ],
    )
    def body(items_h, t1_h, t2_h, m_h, w_h, o_h,
             it_v, t1_v, t2_v, im_v, iw_v, gm, gw, ob, sem_m, sem_w):
        wid = lax.axis_index("s") * 2 + lax.axis_index("c")
        lane = lax.iota(jnp.int32, 16)

        @pl.loop(0, NCHUNK)
        def _chunk(c):
            base = wid * POS_PER_W + c * CHUNK
            pltpu.sync_copy(items_h.at[pl.ds(base, CHUNK)], it_v)
            pltpu.sync_copy(t1_h.at[pl.ds(base, CHUNK)], t1_v)
            pltpu.sync_copy(t2_h.at[pl.ds(base, CHUNK)], t2_v)

            @pl.loop(0, GROUPS)
            def _idx(g):
                it = it_v[pl.ds(g * 16, 16)]
                im_v[pl.ds(g * 16, 16)] = (
                    it * MROWS + lax.shift_right_logical(t1_v[pl.ds(g * 16, 16)], 3))
                iw_v[pl.ds(g * 16, 16)] = (
                    it * WROWS + lax.shift_right_logical(t2_v[pl.ds(g * 16, 16)], 3))

            cm = pltpu.async_copy(m_h.at[im_v], gm, sem_m)
            cw = pltpu.async_copy(w_h.at[iw_v], gw, sem_w)
            cm.wait()
            cw.wait()

            @pl.loop(0, GROUPS)
            def _extract(g):
                rowv = g * 16 + lane
                colm = (t1_v[pl.ds(g * 16, 16)] & 7) * 16
                colw = (t2_v[pl.ds(g * 16, 16)] & 7) * 16
                tgt = rowv * (2 * K)
                for k in range(K):
                    vm = plsc.load_gather(gm, [rowv, colm + k])
                    plsc.store_scatter(ob, [tgt + k], vm)
                    vw = plsc.load_gather(gw, [rowv, colw + k])
                    plsc.store_scatter(ob, [tgt + (K + k)], vw)

            pltpu.sync_copy(ob, o_h.at[pl.ds(base * 2 * K, CHUNK * 2 * K)])

    return body(items, t1, t2, m128, w896)


def kernel(log_seqs, time1_seqs, time2_seqs, month_pop_table, week_pop_table):
    items = log_seqs.reshape(-1).astype(jnp.int32)
    t1 = time1_seqs.reshape(-1).astype(jnp.int32)
    t2 = time2_seqs.reshape(-1).astype(jnp.int32)
    m128 = month_pop_table.T.reshape(NITEM * MROWS, 128)
    w896 = jnp.pad(week_pop_table.T, ((0, 0), (0, WROWS * 128 - NWK * K)))
    w896 = w896.reshape(NITEM * WROWS, 128)
    flat = _sc_gather(items, t1, t2, m128, w896)
    return flat.reshape(B, L, 2 * K)
